# baseline (speedup 1.0000x reference)
import functools

import jax
import jax.numpy as jnp
from jax import lax
from jax.experimental import pallas as pl
from jax.experimental.pallas import tpu as pltpu
from jax.experimental.pallas import tpu_sc as plsc

_K = 1024
_D = 64
_N = 32768
_BETA = 0.25
_BN = 1024


def _tc_body(lat_ref, emb_ref, inds_ref, loss_ref, e_sq_ref):
    @pl.when(pl.program_id(0) == 0)
    def _():
        emb0 = emb_ref[...]
        e_sq_ref[...] = jnp.sum(emb0 * emb0, axis=1)[None, :]

    lat = lat_ref[...]
    emb = emb_ref[...]
    l_sq = jnp.sum(lat * lat, axis=0)[:, None]
    cross = lax.dot_general(
        lat, emb, (((0,), (1,)), ((), ())),
        preferred_element_type=jnp.float32,
        precision=lax.Precision.DEFAULT,
    )
    dist = (l_sq + e_sq_ref[...]) - 2.0 * cross
    _C = 128
    m = dist[:, 0:_C]
    am = jnp.zeros((_BN, _C), jnp.int32)
    for c in range(1, _K // _C):
        d_c = dist[:, c * _C:(c + 1) * _C]
        lt = d_c < m
        m = jnp.where(lt, d_c, m)
        am = jnp.where(lt, c, am)
    mT = m.T
    amT = am.T
    mmin = jnp.min(mT, axis=0)
    lane = lax.broadcasted_iota(jnp.int32, (_C, _BN), 0)
    kfull = amT * _C + lane
    idx = jnp.min(jnp.where(mT <= mmin[None, :], kfull, _K), axis=0)
    inds_ref[...] = idx
    loss_ref[...] = (1.0 + _BETA) * mmin


def _tc_distance_argmin(latents, embedding, rows, row_off):
    grid = rows // _BN
    blk_off = row_off // _BN
    lat_t = jnp.transpose(latents)
    return pl.pallas_call(
        _tc_body,
        grid=(grid,),
        in_specs=[
            pl.BlockSpec((_D, _BN), lambda i: (0, i + blk_off)),
            pl.BlockSpec((_K, _D), lambda i: (0, 0)),
        ],
        out_specs=[
            pl.BlockSpec((_BN,), lambda i: (i,)),
            pl.BlockSpec((_BN,), lambda i: (i,)),
        ],
        out_shape=[
            jax.ShapeDtypeStruct((rows,), jnp.int32),
            jax.ShapeDtypeStruct((rows,), jnp.float32),
        ],
        scratch_shapes=[pltpu.VMEM((1, _K), jnp.float32)],
    )(lat_t, embedding)


_SC_CORES = 2
_SC_SUBCORES = 16
_NW = _SC_CORES * _SC_SUBCORES


@functools.lru_cache(maxsize=None)
def _make_sc_gather(rows):
    bpw = rows // _NW

    @functools.partial(
        pl.kernel,
        mesh=plsc.VectorSubcoreMesh(core_axis_name="c", subcore_axis_name="s"),
        out_type=jax.ShapeDtypeStruct((rows, _D), jnp.float32),
        scratch_types=[
            pltpu.VMEM((bpw,), jnp.int32),
            pltpu.VMEM((bpw, _D), jnp.float32),
            pltpu.SemaphoreType.DMA,
        ],
        compiler_params=pltpu.CompilerParams(use_tc_tiling_on_sc=False),
    )
    def _sc_gather(idx_hbm, table_hbm, out_hbm, idx_v, rows_v, sem):
        wid = lax.axis_index("s") * _SC_CORES + lax.axis_index("c")
        base = wid * bpw
        pltpu.sync_copy(idx_hbm.at[pl.ds(base, bpw)], idx_v)
        pltpu.async_copy(table_hbm.at[idx_v], rows_v, sem).wait()
        pltpu.sync_copy(rows_v, out_hbm.at[pl.ds(base, bpw)])

    return _sc_gather


_FB = 2048


def _fmt_body(in_ref, out_ref):
    x = in_ref[...]
    parts = [x[:, s * _D:(s + 1) * _D] for s in range(8)]
    y = jnp.stack(parts, axis=1)
    out_ref[...] = y.reshape(_FB, _D)


def _tc_format(q_flat):
    q2 = jnp.reshape(q_flat, (_N // 8, 8 * _D))
    return pl.pallas_call(
        _fmt_body,
        grid=(_N // _FB,),
        in_specs=[pl.BlockSpec((_FB // 8, 8 * _D), lambda i: (i, 0))],
        out_specs=pl.BlockSpec((_FB, _D), lambda i: (i, 0)),
        out_shape=jax.ShapeDtypeStruct((_N, _D), jnp.float32),
    )(q2)


def kernel(latents, embedding):
    inds, vq_loss = _tc_distance_argmin(latents, embedding, _N, 0)
    q_lin = _make_sc_gather(_N)(inds, embedding)
    quantized = _tc_format(jnp.reshape(q_lin, (-1,)))
    return quantized, vq_loss


# speedup vs baseline: 1.2381x; 1.2381x over previous
import functools

import jax
import jax.numpy as jnp
from jax import lax
from jax.experimental import pallas as pl
from jax.experimental.pallas import tpu as pltpu
from jax.experimental.pallas import tpu_sc as plsc

_K = 1024
_D = 64
_N = 32768
_BETA = 0.25
_BN = 1024


def _tc_body(lat_ref, emb_ref, inds_ref, loss_ref, e_sq_ref):
    @pl.when(pl.program_id(0) == 0)
    def _():
        emb0 = emb_ref[...]
        e_sq_ref[...] = jnp.sum(emb0 * emb0, axis=1)[None, :]

    lat = lat_ref[...]
    emb = emb_ref[...]
    l_sq = jnp.sum(lat * lat, axis=0)[:, None]
    cross = lax.dot_general(
        lat, emb, (((0,), (1,)), ((), ())),
        preferred_element_type=jnp.float32,
        precision=lax.Precision.DEFAULT,
    )
    _C = 128
    e_sq = e_sq_ref[...]
    m = (l_sq + e_sq[:, 0:_C]) - 2.0 * cross[:, 0:_C]
    am = jnp.zeros((_BN, _C), jnp.int32)
    for c in range(1, _K // _C):
        d_c = (l_sq + e_sq[:, c * _C:(c + 1) * _C]) - 2.0 * cross[:, c * _C:(c + 1) * _C]
        lt = d_c < m
        m = jnp.where(lt, d_c, m)
        am = jnp.where(lt, c, am)
    mT = m.T
    amT = am.T
    mmin = jnp.min(mT, axis=0)
    lane = lax.broadcasted_iota(jnp.int32, (_C, _BN), 0)
    kfull = amT * _C + lane
    idx = jnp.min(jnp.where(mT <= mmin[None, :], kfull, _K), axis=0)
    inds_ref[...] = idx
    loss_ref[...] = (1.0 + _BETA) * mmin


def _tc_distance_argmin(latents, embedding, rows, row_off):
    grid = rows // _BN
    blk_off = row_off // _BN
    lat_t = jnp.transpose(latents)
    return pl.pallas_call(
        _tc_body,
        grid=(grid,),
        in_specs=[
            pl.BlockSpec((_D, _BN), lambda i: (0, i + blk_off)),
            pl.BlockSpec((_K, _D), lambda i: (0, 0)),
        ],
        out_specs=[
            pl.BlockSpec((_BN,), lambda i: (i,)),
            pl.BlockSpec((_BN,), lambda i: (i,)),
        ],
        out_shape=[
            jax.ShapeDtypeStruct((rows,), jnp.int32),
            jax.ShapeDtypeStruct((rows,), jnp.float32),
        ],
        scratch_shapes=[pltpu.VMEM((1, _K), jnp.float32)],
    )(lat_t, embedding)


_SC_CORES = 2
_SC_SUBCORES = 16
_NW = _SC_CORES * _SC_SUBCORES


@functools.lru_cache(maxsize=None)
def _make_sc_gather(rows):
    bpw = rows // _NW

    @functools.partial(
        pl.kernel,
        mesh=plsc.VectorSubcoreMesh(core_axis_name="c", subcore_axis_name="s"),
        out_type=jax.ShapeDtypeStruct((rows, _D), jnp.float32),
        scratch_types=[
            pltpu.VMEM((bpw,), jnp.int32),
            pltpu.VMEM((bpw, _D), jnp.float32),
            pltpu.SemaphoreType.DMA,
        ],
        compiler_params=pltpu.CompilerParams(use_tc_tiling_on_sc=False),
    )
    def _sc_gather(idx_hbm, table_hbm, out_hbm, idx_v, rows_v, sem):
        wid = lax.axis_index("c") * _SC_SUBCORES + lax.axis_index("s")
        base = wid * bpw
        pltpu.sync_copy(idx_hbm.at[pl.ds(base, bpw)], idx_v)
        pltpu.async_copy(table_hbm.at[idx_v], rows_v, sem).wait()
        pltpu.sync_copy(rows_v, out_hbm.at[pl.ds(base, bpw)])

    return _sc_gather


def kernel(latents, embedding):
    inds, vq_loss = _tc_distance_argmin(latents, embedding, _N, 0)
    quantized = _make_sc_gather(_N)(inds, embedding)
    return quantized, vq_loss
